# baseline (speedup 1.0000x reference)
import functools

import jax
import jax.numpy as jnp
from jax import lax
from jax.experimental import pallas as pl
from jax.experimental.pallas import tpu as pltpu
from jax.experimental.pallas import tpu_sc as plsc

_NW = 32
_CHUNK = 128


def _sc_gather(table, idx_3d, n_chunks, D):
    mesh = plsc.VectorSubcoreMesh(core_axis_name="c", subcore_axis_name="s")

    @functools.partial(
        pl.kernel,
        mesh=mesh,
        out_type=jax.ShapeDtypeStruct((_NW, n_chunks, _CHUNK, D), jnp.float32),
        scratch_types=[
            pltpu.VMEM((n_chunks, _CHUNK), jnp.int32),
            pltpu.VMEM((n_chunks, _CHUNK, D), jnp.float32),
            pltpu.SemaphoreType.DMA,
        ],
        compiler_params=pltpu.CompilerParams(use_tc_tiling_on_sc=False),
    )
    def gather_kernel(idx_hbm, table_hbm, out_hbm, idx_v, rows_v, sem):
        wid = lax.axis_index("s") * 2 + lax.axis_index("c")
        pltpu.sync_copy(idx_hbm.at[wid], idx_v)
        copies = []
        for j in range(n_chunks):
            copies.append(
                pltpu.async_copy(table_hbm.at[idx_v.at[j]], rows_v.at[j], sem)
            )
        for c in copies:
            c.wait()
        pltpu.sync_copy(rows_v, out_hbm.at[wid])

    return gather_kernel(idx_3d, table)


def _tc_mlp(emb, gammas, betas, W1, b1, W2, b2, L, BB):
    B, D = emb.shape

    def body(emb_ref, g_ref, bt_ref, w1_ref, b1_ref, w2_ref, b2_ref, out_ref):
        x = emb_ref[...]
        mu = jnp.mean(x, axis=1, keepdims=True)
        xc = x - mu
        var = jnp.mean(xc * xc, axis=1, keepdims=True)
        base = xc * lax.rsqrt(var + 1e-5)
        for i in range(L):
            normed = base * g_ref[i][None, :] + bt_ref[i][None, :]
            h = jnp.dot(normed, w1_ref[i], preferred_element_type=jnp.float32)
            h = jnp.maximum(h + b1_ref[i][None, :], 0.0)
            m = jnp.sum(h * w2_ref[i][None, :], axis=1) + b2_ref[i, 0]
            out_ref[i, :] = jax.nn.sigmoid(m)

    grid = (B // BB,)
    return pl.pallas_call(
        body,
        grid=grid,
        in_specs=[
            pl.BlockSpec((BB, D), lambda i: (i, 0)),
            pl.BlockSpec((L, D), lambda i: (0, 0)),
            pl.BlockSpec((L, D), lambda i: (0, 0)),
            pl.BlockSpec((L, D, D), lambda i: (0, 0, 0)),
            pl.BlockSpec((L, D), lambda i: (0, 0)),
            pl.BlockSpec((L, D), lambda i: (0, 0)),
            pl.BlockSpec((L, 1), lambda i: (0, 0)),
        ],
        out_specs=pl.BlockSpec((L, BB), lambda i: (0, i)),
        out_shape=jax.ShapeDtypeStruct((L, B), jnp.float32),
    )(emb, gammas, betas, W1, b1, W2.reshape(L, D), b2)


def kernel(task_id, table, gammas, betas, W1, b1, W2, b2):
    B = task_id.shape[0]
    D = table.shape[1]
    L = gammas.shape[0]
    n_chunks = B // (_NW * _CHUNK)
    idx_3d = task_id.astype(jnp.int32).reshape(_NW, n_chunks, _CHUNK)
    emb = _sc_gather(table, idx_3d, n_chunks, D).reshape(B, D)
    masks = _tc_mlp(emb, gammas, betas, W1, b1, W2, b2, L, BB=2048)
    return masks.reshape(L, B, 1)


# speedup vs baseline: 1.6610x; 1.6610x over previous
import functools

import jax
import jax.numpy as jnp
from jax import lax
from jax.experimental import pallas as pl
from jax.experimental.pallas import tpu as pltpu
from jax.experimental.pallas import tpu_sc as plsc

_NW = 32


def _sc_gather(table, idx_2d, b_per_w):
    D = table.shape[-1]
    mesh = plsc.VectorSubcoreMesh(core_axis_name="c", subcore_axis_name="s")

    @functools.partial(
        pl.kernel,
        mesh=mesh,
        out_type=jax.ShapeDtypeStruct((_NW, b_per_w, D), jnp.float32),
        scratch_types=[
            pltpu.VMEM((b_per_w,), jnp.int32),
            pltpu.VMEM((b_per_w, D), jnp.float32),
            pltpu.SemaphoreType.DMA,
        ],
    )
    def gather_kernel(idx_hbm, table_hbm, out_hbm, idx_v, rows_v, sem):
        wid = lax.axis_index("s") * 2 + lax.axis_index("c")
        pltpu.sync_copy(idx_hbm.at[wid], idx_v)

        def fire(g, carry):
            iv = idx_v[pl.ds(g * 16, 16)]
            base = g * 16
            for l in range(16):
                r = iv[l]
                pltpu.async_copy(table_hbm.at[pl.ds(r, 1), :],
                                 rows_v.at[pl.ds(base + l, 1), :], sem)
            return carry

        lax.fori_loop(0, b_per_w // 16, fire, 0)

        def drain(i, carry):
            pltpu.make_async_copy(table_hbm.at[pl.ds(0, 1), :],
                                  rows_v.at[pl.ds(i, 1), :], sem).wait()
            return carry

        lax.fori_loop(0, b_per_w, drain, 0, unroll=8)
        pltpu.sync_copy(rows_v, out_hbm.at[wid])

    return gather_kernel(idx_2d, table)


def _tc_mlp(emb, gammas, betas, W1, b1, W2, b2, L, BB):
    B, D = emb.shape

    def body(emb_ref, gm_ref, bt_ref, w1_ref, b1_ref, w2_ref, b2_ref, out_ref):
        x = emb_ref[...]
        mu = jnp.mean(x, axis=1, keepdims=True)
        xc = x - mu
        var = jnp.mean(xc * xc, axis=1, keepdims=True)
        base = xc * lax.rsqrt(var + 1e-5)
        for i in range(L):
            normed = base * gm_ref[i][None, :] + bt_ref[i][None, :]
            h = jnp.dot(normed, w1_ref[i], preferred_element_type=jnp.float32)
            h = jnp.maximum(h + b1_ref[i][None, :], 0.0)
            m = jnp.sum(h * w2_ref[i][None, :], axis=1) + b2_ref[i, 0]
            out_ref[i, :] = jax.nn.sigmoid(m)

    grid = (B // BB,)
    return pl.pallas_call(
        body,
        grid=grid,
        in_specs=[
            pl.BlockSpec((BB, D), lambda i: (i, 0)),
            pl.BlockSpec((L, D), lambda i: (0, 0)),
            pl.BlockSpec((L, D), lambda i: (0, 0)),
            pl.BlockSpec((L, D, D), lambda i: (0, 0, 0)),
            pl.BlockSpec((L, D), lambda i: (0, 0)),
            pl.BlockSpec((L, D), lambda i: (0, 0)),
            pl.BlockSpec((L, 1), lambda i: (0, 0)),
        ],
        out_specs=pl.BlockSpec((L, BB), lambda i: (0, i)),
        out_shape=jax.ShapeDtypeStruct((L, B), jnp.float32),
    )(emb, gammas, betas, W1, b1, W2.reshape(L, D), b2)


def kernel(task_id, table, gammas, betas, W1, b1, W2, b2):
    B = task_id.shape[0]
    D = table.shape[1]
    L = gammas.shape[0]
    b_per_w = B // _NW
    idx_2d = task_id.astype(jnp.int32).reshape(_NW, b_per_w)
    emb = _sc_gather(table, idx_2d, b_per_w).reshape(B, D)
    masks = _tc_mlp(emb, gammas, betas, W1, b1, W2, b2, L, BB=2048)
    return masks.reshape(L, B, 1)


# speedup vs baseline: 2.4495x; 1.4747x over previous
import functools

import jax
import jax.numpy as jnp
from jax import lax
from jax.experimental import pallas as pl
from jax.experimental.pallas import tpu as pltpu
from jax.experimental.pallas import tpu_sc as plsc

_NW = 32
_NBUF = 8


def _sc_gather_cols(tableT, idx_2d, b_per_w):
    D, V = tableT.shape
    max_start = ((V - 128) // 128) * 128
    mesh = plsc.VectorSubcoreMesh(core_axis_name="c", subcore_axis_name="s")

    @functools.partial(
        pl.kernel,
        mesh=mesh,
        out_type=jax.ShapeDtypeStruct((_NW, D, b_per_w), jnp.float32),
        scratch_types=[
            pltpu.VMEM((b_per_w,), jnp.int32),
            pltpu.VMEM((_NBUF, D, 128), jnp.float32),
            pltpu.VMEM((D, b_per_w), jnp.float32),
            pltpu.SemaphoreType.DMA,
        ],
        compiler_params=pltpu.CompilerParams(needs_layout_passes=False),
    )
    def gather_kernel(idx_hbm, table_hbm, out_hbm, idx_v, fetch_v, cols_v,
                      sem):
        wid = lax.axis_index("s") * 2 + lax.axis_index("c")
        pltpu.sync_copy(idx_hbm.at[wid], idx_v)
        rows = [jnp.arange(p * 16, p * 16 + 16, dtype=jnp.int32)
                for p in range(D // 16)]

        def wave(g, carry):
            iv = idx_v[pl.ds(g * 16, 16)]
            for h in range(2):
                copies = []
                starts = []
                for q in range(_NBUF):
                    s = h * _NBUF + q
                    r = iv[s]
                    jt = lax.shift_right_logical(r, 7) * 128
                    start = jnp.minimum(jt, max_start)
                    starts.append(start)
                    copies.append(pltpu.async_copy(
                        table_hbm.at[:, pl.ds(pl.multiple_of(start, 128),
                                              128)],
                        fetch_v.at[q], sem))
                for q in range(_NBUF):
                    s = h * _NBUF + q
                    copies[q].wait()
                    r = iv[s]
                    loff = jnp.minimum(r - starts[q], 127)
                    cvec = jnp.zeros((16,), jnp.int32) + loff
                    ovec = jnp.zeros((16,), jnp.int32) + (g * 16 + s)
                    for p in range(D // 16):
                        vals = plsc.load_gather(fetch_v.at[q], [rows[p], cvec])
                        plsc.store_scatter(cols_v, [rows[p], ovec], vals)
            return carry

        lax.fori_loop(0, b_per_w // 16, wave, 0)
        pltpu.sync_copy(cols_v, out_hbm.at[wid])

    return gather_kernel(idx_2d, tableT)


def _tc_mlp_t(embT, idx_3d, tail, gammas, betas, W1, b1, W2, b2, L, t0):
    NW, D, bw = embT.shape
    B = NW * bw
    n_tail = tail.shape[0]

    def body(e_ref, i_ref, tl_ref, gm_ref, bt_ref, w1_ref, b1_ref, w2_ref,
             b2_ref, out_ref):
        x = e_ref[0]
        toff = i_ref[0, 0] - t0
        sub = jax.lax.broadcasted_iota(jnp.int32, (n_tail, bw), 0)
        onehot = jnp.where(sub == toff[None, :], 1.0, 0.0)
        fix = lax.dot_general(tl_ref[...], onehot, (((0,), (0,)), ((), ())),
                              preferred_element_type=jnp.float32)
        x = jnp.where((toff >= 0)[None, :], fix, x)
        mu = jnp.mean(x, axis=0, keepdims=True)
        xc = x - mu
        var = jnp.mean(xc * xc, axis=0, keepdims=True)
        base = xc * lax.rsqrt(var + 1e-5)
        for i in range(L):
            normed = base * gm_ref[i][:, None] + bt_ref[i][:, None]
            h = lax.dot_general(w1_ref[i], normed,
                                (((0,), (0,)), ((), ())),
                                preferred_element_type=jnp.float32)
            h = jnp.maximum(h + b1_ref[i][:, None], 0.0)
            m = jnp.sum(h * w2_ref[i][:, None], axis=0) + b2_ref[i, 0]
            out_ref[i, :] = jax.nn.sigmoid(m)

    grid = (NW,)
    return pl.pallas_call(
        body,
        grid=grid,
        in_specs=[
            pl.BlockSpec((1, D, bw), lambda w: (w, 0, 0)),
            pl.BlockSpec((1, 1, bw), lambda w: (w, 0, 0)),
            pl.BlockSpec((n_tail, D), lambda w: (0, 0)),
            pl.BlockSpec((L, D), lambda w: (0, 0)),
            pl.BlockSpec((L, D), lambda w: (0, 0)),
            pl.BlockSpec((L, D, D), lambda w: (0, 0, 0)),
            pl.BlockSpec((L, D), lambda w: (0, 0)),
            pl.BlockSpec((L, D), lambda w: (0, 0)),
            pl.BlockSpec((L, 1), lambda w: (0, 0)),
        ],
        out_specs=pl.BlockSpec((L, bw), lambda w: (0, w)),
        out_shape=jax.ShapeDtypeStruct((L, B), jnp.float32),
    )(embT, idx_3d, tail, gammas, betas, W1, b1, W2.reshape(L, D), b2)


def kernel(task_id, table, gammas, betas, W1, b1, W2, b2):
    B = task_id.shape[0]
    V, D = table.shape
    L = gammas.shape[0]
    b_per_w = B // _NW
    idx_2d = task_id.astype(jnp.int32).reshape(_NW, b_per_w)
    tableT = table.T
    t0 = ((V - 128) // 128) * 128 + 128
    tail = table[t0:, :]
    embT = _sc_gather_cols(tableT, idx_2d, b_per_w)
    masks = _tc_mlp_t(embT, idx_2d.reshape(_NW, 1, b_per_w), tail, gammas, betas, W1, b1, W2, b2, L,
                      t0)
    return masks.reshape(L, B, 1)
